# Initial kernel scaffold; baseline (speedup 1.0000x reference)
import jax
import jax.numpy as jnp
from jax.experimental import pallas as pl


def kernel(source_recon_pc, target_recon_pc, source_inv_f, target_inv_f, source_keypoints, target_keypoints, influence_param, template_vertices, template_faces, W1, b1, W2, b2, W3, b3, W4, b4):
    raise NotImplementedError("write your pallas kernel here")


# speedup vs baseline: 2.0561x; 2.0561x over previous
import functools

import numpy as np
import jax
import jax.numpy as jnp
from jax.experimental import pallas as pl

_DIST = 0.4
_STEP = 0.01
_ITERS = 100
_TILE_N = 512

_HI = jax.lax.Precision.HIGHEST


def _asin(x):
    return 2.0 * jnp.arctan2(x, 1.0 + jnp.sqrt((1.0 - x) * (1.0 + x)))


def _mm(a, b):
    return jax.lax.dot_general(a, b, (((1,), (0,)), ((), ())),
                               precision=_HI, preferred_element_type=jnp.float32)


def _mm_nt(a, b):
    return jax.lax.dot_general(a, b, (((1,), (1,)), ((), ())),
                               precision=_HI, preferred_element_type=jnp.float32)


def _cage_kernel(tv_ref, x_ref, cage_ref):
    tv = tv_ref[...]
    x = x_ref[0]
    px, py, pz = x[0:1], x[1:2], x[2:3]
    dist = np.float32(_DIST)
    step = np.float32(_STEP)

    def body(_, c):
        dx = c[:, 0:1] - px
        dy = c[:, 1:2] - py
        dz = c[:, 2:3] - pz
        m = jnp.min(dx * dx + dy * dy + dz * dz, axis=1, keepdims=True)
        do = (jnp.sqrt(m) > dist).astype(jnp.float32)
        return c + (step * (-c)) * do

    cage_ref[0] = jax.lax.fori_loop(0, _ITERS, body, tv)


def _influence_kernel(sif_ref, tif_ref, skp_ref, tkp_ref, ip_ref, cage3_ref,
                      w1_ref, b1_ref, w2_ref, b2_ref, w3_ref, b3_ref,
                      w4_ref, b4_ref, off_ref, nc_ref, n_inf):
    comb = jnp.concatenate([sif_ref[0], tif_ref[0]], axis=1)
    h = jax.nn.relu(_mm_nt(comb, w1_ref[...]) + b1_ref[...])
    h = jax.nn.relu(_mm_nt(h, w2_ref[...]) + b2_ref[...])
    h = jax.nn.relu(_mm_nt(h, w3_ref[...]) + b3_ref[...])
    off = _mm_nt(h, w4_ref[...]) + b4_ref[...]
    off_ref[0] = off

    cage3 = cage3_ref[0]
    kp = skp_ref[0]
    K, C = kp.shape[0], cage3.shape[1]
    cx, cy, cz = cage3[0:1], cage3[1:2], cage3[2:3]
    kx, ky, kz = kp[:, 0:1], kp[:, 1:2], kp[:, 2:3]
    ddx, ddy, ddz = kx - cx, ky - cy, kz - cz
    d = ddx * ddx + ddy * ddy + ddz * ddz

    di = d[:, :, None]
    dj = d[:, None, :]
    ii = jax.lax.broadcasted_iota(jnp.int32, (K, C, C), 1)
    jj = jax.lax.broadcasted_iota(jnp.int32, (K, C, C), 2)
    rank = jnp.sum((dj < di).astype(jnp.float32)
                   + ((dj == di) & (jj < ii)).astype(jnp.float32), axis=2)
    sel = (rank == np.float32(n_inf - 1)).astype(jnp.float32)
    thr = jnp.sum(d * sel, axis=1, keepdims=True)
    keep = (d <= thr).astype(jnp.float32)

    infl = (ip_ref[...] + off) * keep
    kp_off = tkp_ref[0] - kp
    co = _mm(jnp.swapaxes(infl, 0, 1), kp_off)
    nc_ref[0] = jnp.swapaxes(cage3, 0, 1) + co


def _mvc_kernel(x_ref, cg_ref, nc_ref, st_ref, wm_ref, def_ref):
    x = x_ref[0]
    cg = cg_ref[0]
    St = st_ref[...]
    F = cg.shape[0] // 3

    px, py, pz = x[0:1], x[1:2], x[2:3]
    dx = cg[:, 0:1] - px
    dy = cg[:, 1:2] - py
    dz = cg[:, 2:3] - pz
    df = jnp.sqrt(dx * dx + dy * dy + dz * dz + 1e-12)
    ux, uy, uz = dx / df, dy / df, dz / df

    def rollm(a):
        return jnp.concatenate([a[F:], a[:F]], axis=0)

    def rollp(a):
        return jnp.concatenate([a[2 * F:], a[:2 * F]], axis=0)

    ex = rollm(ux) - rollp(ux)
    ey = rollm(uy) - rollp(uy)
    ez = rollm(uz) - rollp(uz)
    l = jnp.sqrt(ex * ex + ey * ey + ez * ez + 1e-12)
    sh = jnp.clip(l / 2.0, 0.0, 1.0 - 1e-7)
    theta = 2.0 * _asin(sh)
    hh = (theta[:F] + theta[F:2 * F] + theta[2 * F:]) / 2.0
    sin_h = jnp.sin(hh)
    sin_h3 = jnp.concatenate([sin_h, sin_h, sin_h], axis=0)
    h3 = jnp.concatenate([hh, hh, hh], axis=0)
    sin_hmt = jnp.sin(h3 - theta)
    sin_t = jnp.sin(theta)

    cs = 2.0 * sin_h3 * sin_hmt / (rollm(sin_t) * rollp(sin_t) + 1e-8) - 1.0
    cs = jnp.clip(cs, -1.0, 1.0)

    u0x, u1x, u2x = ux[:F], ux[F:2 * F], ux[2 * F:]
    u0y, u1y, u2y = uy[:F], uy[F:2 * F], uy[2 * F:]
    u0z, u1z, u2z = uz[:F], uz[F:2 * F], uz[2 * F:]
    det = (u0x * (u1y * u2z - u1z * u2y)
           + u0y * (u1z * u2x - u1x * u2z)
           + u0z * (u1x * u2y - u1y * u2x))
    sgn = jnp.sign(det)
    sgn3 = jnp.concatenate([sgn, sgn, sgn], axis=0)
    s = sgn3 * jnp.sqrt(jnp.clip(1.0 - cs * cs, 1e-10, None))

    w = (theta - rollm(cs) * rollp(theta) - rollp(cs) * rollm(theta)) / (
        df * rollm(sin_t) * rollp(s) + 1e-8)

    wm = _mm(St, w)
    wm = wm / (jnp.sum(wm, axis=0, keepdims=True) + 1e-8)
    wm_ref[0] = wm
    def_ref[0] = _mm(nc_ref[0], wm)


def kernel(source_recon_pc, target_recon_pc, source_inv_f, target_inv_f,
           source_keypoints, target_keypoints, influence_param,
           template_vertices, template_faces, W1, b1, W2, b2, W3, b3, W4, b4):
    f32 = jnp.float32
    B, _, N = source_recon_pc.shape
    K = source_keypoints.shape[1]
    C = template_vertices.shape[2]
    n_inf = max(5, C // K)
    tile = min(_TILE_N, N)

    tv = jnp.swapaxes(template_vertices[0], 0, 1)

    cage_T = pl.pallas_call(
        _cage_kernel,
        grid=(B,),
        in_specs=[pl.BlockSpec((C, 3), lambda b: (0, 0)),
                  pl.BlockSpec((1, 3, N), lambda b: (b, 0, 0))],
        out_specs=pl.BlockSpec((1, C, 3), lambda b: (b, 0, 0)),
        out_shape=jax.ShapeDtypeStruct((B, C, 3), f32),
    )(tv, source_recon_pc)

    cage3 = jnp.swapaxes(cage_T, 1, 2)
    sif = source_inv_f[:, None, :]
    tif = target_inv_f[:, None, :]
    Fin = sif.shape[2]
    H1, H2, H3 = W1.shape[0], W2.shape[0], W3.shape[0]
    off, new_cage_T = pl.pallas_call(
        functools.partial(_influence_kernel, n_inf=n_inf),
        grid=(B,),
        in_specs=[pl.BlockSpec((1, 1, Fin), lambda b: (b, 0, 0)),
                  pl.BlockSpec((1, 1, Fin), lambda b: (b, 0, 0)),
                  pl.BlockSpec((1, K, 3), lambda b: (b, 0, 0)),
                  pl.BlockSpec((1, K, 3), lambda b: (b, 0, 0)),
                  pl.BlockSpec((K, C), lambda b: (0, 0)),
                  pl.BlockSpec((1, 3, C), lambda b: (b, 0, 0)),
                  pl.BlockSpec(W1.shape, lambda b: (0, 0)),
                  pl.BlockSpec((1, H1), lambda b: (0, 0)),
                  pl.BlockSpec(W2.shape, lambda b: (0, 0)),
                  pl.BlockSpec((1, H2), lambda b: (0, 0)),
                  pl.BlockSpec(W3.shape, lambda b: (0, 0)),
                  pl.BlockSpec((1, H3), lambda b: (0, 0)),
                  pl.BlockSpec(W4.shape, lambda b: (0, 0)),
                  pl.BlockSpec((1, C), lambda b: (0, 0))],
        out_specs=[pl.BlockSpec((1, 1, C), lambda b: (b, 0, 0)),
                   pl.BlockSpec((1, C, 3), lambda b: (b, 0, 0))],
        out_shape=[jax.ShapeDtypeStruct((B, 1, C), f32),
                   jax.ShapeDtypeStruct((B, C, 3), f32)],
    )(sif, tif, source_keypoints, target_keypoints, influence_param, cage3,
      W1, b1[None, :], W2, b2[None, :], W3, b3[None, :], W4, b4[None, :])

    faces = template_faces[0]
    Fc = faces.shape[0]
    idx = jnp.concatenate([faces[:, 0], faces[:, 1], faces[:, 2]])
    cage_g = cage_T[:, idx, :]
    St = jnp.swapaxes(jax.nn.one_hot(idx, C, dtype=f32), 0, 1)
    new_cage = jnp.swapaxes(new_cage_T, 1, 2)

    wm_T, deformed = pl.pallas_call(
        _mvc_kernel,
        grid=(B, N // tile),
        in_specs=[pl.BlockSpec((1, 3, tile), lambda b, n: (b, 0, n)),
                  pl.BlockSpec((1, 3 * Fc, 3), lambda b, n: (b, 0, 0)),
                  pl.BlockSpec((1, 3, C), lambda b, n: (b, 0, 0)),
                  pl.BlockSpec((C, 3 * Fc), lambda b, n: (0, 0))],
        out_specs=[pl.BlockSpec((1, C, tile), lambda b, n: (b, 0, n)),
                   pl.BlockSpec((1, 3, tile), lambda b, n: (b, 0, n))],
        out_shape=[jax.ShapeDtypeStruct((B, C, N), f32),
                   jax.ShapeDtypeStruct((B, 3, N), f32)],
    )(source_recon_pc, cage_g, new_cage, St)

    Wm = jnp.swapaxes(wm_T, 1, 2)
    return (cage_T, new_cage_T, deformed, Wm, off)
